# baseline (speedup 1.0000x reference)
import functools

import jax
import jax.numpy as jnp
from jax import lax
from jax.experimental import pallas as pl
from jax.experimental.pallas import tpu as pltpu
from jax.experimental.pallas import tpu_sc as plsc

NC = 2
NS = 16
NW = NC * NS
LANES = 16


def _make_emb_kernel(B, S, D, N, SW, C, SCH):
    mesh = plsc.VectorSubcoreMesh(
        core_axis_name="c", subcore_axis_name="s",
        num_cores=NC, num_subcores=NS,
    )
    NR = SCH * B

    @functools.partial(
        pl.kernel,
        mesh=mesh,
        out_type=jax.ShapeDtypeStruct((N, D), jnp.float32),
        scratch_types=[
            pltpu.VMEM((B, SW), jnp.int32),
            pltpu.VMEM((2, C, D), jnp.float32),
            pltpu.VMEM((2, C, D), jnp.float32),
            [pltpu.SemaphoreType.DMA] * 2,
            [pltpu.SemaphoreType.DMA] * 2,
            [pltpu.SemaphoreType.DMA] * 2,
        ],
    )
    def emb_kernel(idx_hbm, pe_hbm, table_hbm, out_hbm, idx_v, pe_v, rows_v,
                   g_sems, s_sems, p_sems):
        wid = lax.axis_index("s") * NC + lax.axis_index("c")
        s_base = wid * SW
        pltpu.sync_copy(idx_hbm.at[wid], idx_v)

        def gather(r):
            sc, b = divmod(r, B)
            buf = r % 2
            return pltpu.async_copy(
                table_hbm.at[idx_v.at[b, pl.ds(sc * C, C)]],
                rows_v.at[buf], g_sems[buf],
            )

        def pe_load(sc):
            p0 = pl.multiple_of(s_base + sc * C, 8)
            return pltpu.async_copy(
                pe_hbm.at[pl.ds(p0, C)], pe_v.at[sc % 2], p_sems[sc % 2],
            )

        def store(r):
            sc, b = divmod(r, B)
            buf = r % 2
            r0 = pl.multiple_of(b * S + s_base + sc * C, 8)
            return pltpu.async_copy(
                rows_v.at[buf], out_hbm.at[pl.ds(r0, C)], s_sems[buf],
            )

        pe_d = [None] * SCH
        g_d = [None] * NR
        st_d = [None] * NR

        pe_d[0] = pe_load(0)
        g_d[0] = gather(0)
        g_d[1] = gather(1)

        for r in range(NR):
            sc, b = divmod(r, B)
            buf = r % 2
            if b == 0:
                pe_d[sc].wait()
                if sc + 1 < SCH:
                    pe_d[sc + 1] = pe_load(sc + 1)
            if 1 <= r and r + 1 < NR:
                st_d[r - 1].wait()
                g_d[r + 1] = gather(r + 1)
            g_d[r].wait()

            def row_body(i, carry):
                for k in range(D // LANES):
                    sl = pl.ds(k * LANES, LANES)
                    plsc.addupdate(rows_v.at[buf, i, sl], pe_v[sc % 2, i, sl])
                return carry

            lax.fori_loop(0, C, row_body, 0)
            st_d[r] = store(r)

        st_d[NR - 2].wait()
        st_d[NR - 1].wait()

    return emb_kernel


def kernel(x, table, pos_encoding):
    B, S = x.shape
    D = table.shape[1]
    N = B * S
    SW = S // NW
    C = 32
    SCH = SW // C

    idx = x.astype(jnp.int32).reshape(B, NW, SW).transpose(1, 0, 2)
    pe = pos_encoding.reshape(S, D).astype(jnp.float32)

    emb = _make_emb_kernel(B, S, D, N, SW, C, SCH)
    out = emb(idx, pe, table)
    return out.reshape(B, S, D)


# speedup vs baseline: 1.4806x; 1.4806x over previous
import functools

import jax
import jax.numpy as jnp
from jax import lax
from jax.experimental import pallas as pl
from jax.experimental.pallas import tpu as pltpu
from jax.experimental.pallas import tpu_sc as plsc

NC = 2
NS = 16
NW = NC * NS
LANES = 16


def _make_emb_kernel(B, S, D, N, SW, C, SCH):
    mesh = plsc.VectorSubcoreMesh(
        core_axis_name="c", subcore_axis_name="s",
        num_cores=NC, num_subcores=NS,
    )
    NR = SCH * B

    @functools.partial(
        pl.kernel,
        mesh=mesh,
        out_type=jax.ShapeDtypeStruct((N, D), jnp.float32),
        scratch_types=[
            pltpu.VMEM((B, SW), jnp.int32),
            pltpu.VMEM((2, C, D), jnp.float32),
            pltpu.VMEM((2, C, D), jnp.float32),
            [pltpu.SemaphoreType.DMA] * 2,
            [pltpu.SemaphoreType.DMA] * 2,
            [pltpu.SemaphoreType.DMA] * 2,
        ],
    )
    def emb_kernel(idx_hbm, pe_hbm, table_hbm, out_hbm, idx_v, pe_v, rows_v,
                   g_sems, s_sems, p_sems):
        wid = lax.axis_index("s") * NC + lax.axis_index("c")
        s_base = wid * SW
        pltpu.sync_copy(idx_hbm.at[wid], idx_v)

        def gather(r):
            sc, b = divmod(r, B)
            buf = r % 2
            return pltpu.async_copy(
                table_hbm.at[idx_v.at[b, pl.ds(sc * C, C)]],
                rows_v.at[buf], g_sems[buf],
            )

        def pe_load(sc):
            p0 = pl.multiple_of(s_base + sc * C, 8)
            return pltpu.async_copy(
                pe_hbm.at[pl.ds(p0, C)], pe_v.at[sc % 2], p_sems[sc % 2],
            )

        def store(r):
            sc, b = divmod(r, B)
            buf = r % 2
            r0 = pl.multiple_of(b * S + s_base + sc * C, 8)
            return pltpu.async_copy(
                rows_v.at[buf], out_hbm.at[pl.ds(r0, C)], s_sems[buf],
            )

        pe_d = [None] * SCH
        g_d = [None] * NR
        st_d = [None] * NR

        pe_d[0] = pe_load(0)
        g_d[0] = gather(0)
        g_d[1] = gather(1)

        for r in range(NR):
            sc, b = divmod(r, B)
            buf = r % 2
            if b == 0:
                pe_d[sc].wait()
                if sc + 1 < SCH:
                    pe_d[sc + 1] = pe_load(sc + 1)
            if 1 <= r and r + 1 < NR:
                st_d[r - 1].wait()
                g_d[r + 1] = gather(r + 1)
            g_d[r].wait()

            def row_body(i, carry):
                @plsc.parallel_loop(0, D, step=LANES, unroll=4)
                def lane_body(k):
                    sl = pl.ds(k, LANES)
                    plsc.addupdate(rows_v.at[buf, i, sl], pe_v[sc % 2, i, sl])

                return carry

            lax.fori_loop(0, C, row_body, 0)

            st_d[r] = store(r)

        st_d[NR - 2].wait()
        st_d[NR - 1].wait()

    return emb_kernel


def kernel(x, table, pos_encoding):
    B, S = x.shape
    D = table.shape[1]
    N = B * S
    SW = S // NW
    C = 32
    SCH = SW // C

    idx = x.astype(jnp.int32).reshape(B, NW, SW).transpose(1, 0, 2)
    pe = pos_encoding.reshape(S, D).astype(jnp.float32)

    emb = _make_emb_kernel(B, S, D, N, SW, C, SCH)
    out = emb(idx, pe, table)
    return out.reshape(B, S, D)
